# Initial kernel scaffold; baseline (speedup 1.0000x reference)
import jax
import jax.numpy as jnp
from jax.experimental import pallas as pl


def kernel(inputs, embeddings):
    raise NotImplementedError("write your pallas kernel here")


# speedup vs baseline: 7.8537x; 7.8537x over previous
import jax
import jax.numpy as jnp
from jax.experimental import pallas as pl
from jax.experimental.pallas import tpu as pltpu

_V = 1000
_VP = 1024
_QB = 128


def _nn_body(q_ref, e_ref, out_ref):
    q = q_ref[...]
    e = e_ref[...]
    e2 = jnp.sum(e * e, axis=1)
    qe = jax.lax.dot_general(
        e, q, (((1,), (1,)), ((), ())),
        preferred_element_type=jnp.float32,
        precision=jax.lax.Precision.HIGHEST,
    )
    dist = e2[:, None] - 2.0 * qe
    row = jax.lax.broadcasted_iota(jnp.int32, dist.shape, 0)
    dist = jnp.where(row < _V, dist, jnp.inf)
    idx = jnp.argmin(dist, axis=0).astype(jnp.int32)
    out_ref[0, 0, :] = idx


def kernel(inputs, embeddings):
    B, S, D = inputs.shape
    nq = B * S
    nqp = ((nq + _QB - 1) // _QB) * _QB
    q = jnp.zeros((nqp, D), jnp.float32).at[:nq].set(inputs.reshape(nq, D))
    e_pad = jnp.zeros((_VP, D), jnp.float32).at[:_V].set(embeddings)
    grid = nqp // _QB
    out = pl.pallas_call(
        _nn_body,
        grid=(grid,),
        in_specs=[
            pl.BlockSpec((_QB, D), lambda i: (i, 0)),
            pl.BlockSpec((_VP, D), lambda i: (0, 0)),
        ],
        out_specs=pl.BlockSpec((1, 1, _QB), lambda i: (i, 0, 0)),
        out_shape=jax.ShapeDtypeStruct((grid, 1, _QB), jnp.int32),
    )(q, e_pad)
    return out.reshape(nqp)[:nq].reshape(B, S)
